# Initial kernel scaffold; baseline (speedup 1.0000x reference)
import jax
import jax.numpy as jnp
from jax.experimental import pallas as pl


def kernel(preds, target, batch_map):
    raise NotImplementedError("write your pallas kernel here")


# speedup vs baseline: 47.0940x; 47.0940x over previous
import functools

import jax
import jax.numpy as jnp
from jax import lax
from jax.experimental import pallas as pl
from jax.experimental.pallas import tpu as pltpu
from jax.experimental.pallas import tpu_sc as plsc

N = 6_400_000
NSEG = 512
NC = 2
NS = 16
L = 16
NW = NC * NS
CHUNK = N // NW
PIECE = 8_000
NPIECES = CHUNK // PIECE
VPP = PIECE // L


def _sc_body(p_hbm, t_hbm, g_hbm, out_hbm,
             pv0, pv1, tv0, tv1, gv0, gv1,
             acc_pp, acc_tt, acc_pt,
             sh_pp, sh_tt, sh_pt,
             sem0, sem1):
    c = lax.axis_index("c")
    s = lax.axis_index("s")
    wid = c * NS + s
    base = wid * CHUNK

    lane = lax.iota(jnp.int32, L)
    zero16 = jnp.zeros((L,), jnp.float32)
    zidx = jnp.zeros((L,), jnp.int32)

    def zbody(i, carry):
        r = i // (NSEG // L)
        k = i % (NSEG // L)
        acc_pp[r, pl.ds(k * L, L)] = zero16
        acc_tt[r, pl.ds(k * L, L)] = zero16
        acc_pt[r, pl.ds(k * L, L)] = zero16
        return carry
    lax.fori_loop(0, L * (NSEG // L), zbody, 0)

    @pl.when(s == 0)
    def _():
        pltpu.sync_copy(acc_pp.at[pl.ds(0, 1)], sh_pp)
        pltpu.sync_copy(acc_tt.at[pl.ds(0, 1)], sh_tt)
        pltpu.sync_copy(acc_pt.at[pl.ds(0, 1)], sh_pt)
    plsc.subcore_barrier()

    pvs, tvs, gvs, sems = [pv0, pv1], [tv0, tv1], [gv0, gv1], [sem0, sem1]

    def fire(slot, j):
        off = pl.multiple_of(base + j * PIECE, PIECE)
        return [
            pltpu.async_copy(p_hbm.at[pl.ds(off, PIECE)], pvs[slot], sems[slot]),
            pltpu.async_copy(t_hbm.at[pl.ds(off, PIECE)], tvs[slot], sems[slot]),
            pltpu.async_copy(g_hbm.at[pl.ds(off, PIECE)], gvs[slot], sems[slot]),
        ]

    def compute(slot):
        pv, tv, gv = pvs[slot], tvs[slot], gvs[slot]
        def cbody(i, carry):
            o = pl.ds(pl.multiple_of(i * L, L), L)
            p = pv[o]
            t = tv[o]
            g = gv[o]
            plsc.addupdate_scatter(acc_pp, [lane, g], p * p)
            plsc.addupdate_scatter(acc_tt, [lane, g], t * t)
            plsc.addupdate_scatter(acc_pt, [lane, g], p * t)
            return carry
        lax.fori_loop(0, VPP, cbody, 0)

    handles = [None, None]
    handles[0] = fire(0, 0)
    for j in range(NPIECES):
        slot = j & 1
        if j + 1 < NPIECES:
            handles[slot ^ 1] = fire(slot ^ 1, j + 1)
        for h in handles[slot]:
            h.wait()
        compute(slot)

    pltpu.sync_copy(acc_pp, sh_pp.at[zidx], add=True)
    pltpu.sync_copy(acc_tt, sh_tt.at[zidx], add=True)
    pltpu.sync_copy(acc_pt, sh_pt.at[zidx], add=True)
    plsc.subcore_barrier()

    @pl.when(s == 0)
    def _():
        pltpu.sync_copy(sh_pp, out_hbm.at[pl.ds(c * 3 + 0, 1)])
        pltpu.sync_copy(sh_tt, out_hbm.at[pl.ds(c * 3 + 1, 1)])
        pltpu.sync_copy(sh_pt, out_hbm.at[pl.ds(c * 3 + 2, 1)])


_segment_sums = pl.kernel(
    _sc_body,
    out_type=jax.ShapeDtypeStruct((NC * 3, NSEG), jnp.float32),
    mesh=plsc.VectorSubcoreMesh(core_axis_name="c", subcore_axis_name="s"),
    compiler_params=pltpu.CompilerParams(use_tc_tiling_on_sc=False,
                                         needs_layout_passes=False),
    scratch_types=[
        pltpu.VMEM((PIECE,), jnp.float32),
        pltpu.VMEM((PIECE,), jnp.float32),
        pltpu.VMEM((PIECE,), jnp.float32),
        pltpu.VMEM((PIECE,), jnp.float32),
        pltpu.VMEM((PIECE,), jnp.int32),
        pltpu.VMEM((PIECE,), jnp.int32),
        pltpu.VMEM((L, NSEG), jnp.float32),
        pltpu.VMEM((L, NSEG), jnp.float32),
        pltpu.VMEM((L, NSEG), jnp.float32),
        pltpu.VMEM_SHARED((1, NSEG), jnp.float32),
        pltpu.VMEM_SHARED((1, NSEG), jnp.float32),
        pltpu.VMEM_SHARED((1, NSEG), jnp.float32),
        pltpu.SemaphoreType.DMA,
        pltpu.SemaphoreType.DMA,
    ],
)


def kernel(preds, target, batch_map):
    parts = _segment_sums(preds, target, batch_map.astype(jnp.int32))
    parts = parts.reshape(NC, 3, NSEG).sum(axis=0)
    pp, tt, pt = parts[0], parts[1], parts[2]
    eps = 1e-12
    cosine = pt / jnp.maximum(jnp.sqrt(pp) * jnp.sqrt(tt), eps)
    return jnp.mean(1.0 - cosine)


# speedup vs baseline: 155.6098x; 3.3042x over previous
import functools

import jax
import jax.numpy as jnp
from jax import lax
from jax.experimental import pallas as pl
from jax.experimental.pallas import tpu as pltpu
from jax.experimental.pallas import tpu_sc as plsc

N = 6_400_000
NSEG = 512
NC = 2
NS = 16
L = 16
NW = NC * NS
CHUNK = N // NW
PIECE = 8_000
NPIECES = CHUNK // PIECE
VPP = PIECE // L
SPT = NSEG // NS


def _sc_body(p_hbm, t_hbm, g_hbm, out_hbm,
             pv0, pv1, tv0, tv1, gv0, gv1,
             acc_pp, acc_tt, acc_pt,
             idx_v,
             sh_pp, sh_tt, sh_pt,
             sem0, sem1):
    c = lax.axis_index("c")
    s = lax.axis_index("s")
    wid = c * NS + s
    base = wid * CHUNK

    lane = lax.iota(jnp.int32, L)
    zero16 = jnp.zeros((L,), jnp.float32)

    def zbody(i, carry):
        acc_pp[i] = zero16
        acc_tt[i] = zero16
        acc_pt[i] = zero16
        return carry
    lax.fori_loop(0, NSEG, zbody, 0)

    def ibody(j, carry):
        idx_v[pl.ds(j * L, L)] = lane + j * L
        return carry
    lax.fori_loop(0, NSEG // L, ibody, 0)

    @pl.when(s == 0)
    def _():
        pltpu.sync_copy(acc_pp, sh_pp)
        pltpu.sync_copy(acc_tt, sh_tt)
        pltpu.sync_copy(acc_pt, sh_pt)
    plsc.subcore_barrier()

    pvs, tvs, gvs, sems = [pv0, pv1], [tv0, tv1], [gv0, gv1], [sem0, sem1]

    def fire(slot, j):
        off = pl.multiple_of(base + j * PIECE, PIECE)
        return [
            pltpu.async_copy(p_hbm.at[pl.ds(off, PIECE)], pvs[slot], sems[slot]),
            pltpu.async_copy(t_hbm.at[pl.ds(off, PIECE)], tvs[slot], sems[slot]),
            pltpu.async_copy(g_hbm.at[pl.ds(off, PIECE)], gvs[slot], sems[slot]),
        ]

    def compute(slot):
        pv, tv, gv = pvs[slot], tvs[slot], gvs[slot]
        def cbody(i, carry):
            o = pl.ds(pl.multiple_of(i * L, L), L)
            p = pv[o]
            t = tv[o]
            g = gv[o]
            plsc.addupdate_scatter(acc_pp, [g, lane], p * p)
            plsc.addupdate_scatter(acc_tt, [g, lane], t * t)
            plsc.addupdate_scatter(acc_pt, [g, lane], p * t)
            return carry
        lax.fori_loop(0, VPP, cbody, 0)

    handles = [None, None]
    handles[0] = fire(0, 0)
    for j in range(NPIECES):
        slot = j & 1
        if j + 1 < NPIECES:
            handles[slot ^ 1] = fire(slot ^ 1, j + 1)
        for h in handles[slot]:
            h.wait()
        compute(slot)

    pltpu.sync_copy(acc_pp, sh_pp.at[idx_v], add=True)
    pltpu.sync_copy(acc_tt, sh_tt.at[idx_v], add=True)
    pltpu.sync_copy(acc_pt, sh_pt.at[idx_v], add=True)
    plsc.subcore_barrier()

    sbase = s * SPT
    for q, sh in enumerate((sh_pp, sh_tt, sh_pt)):
        pltpu.sync_copy(sh.at[pl.ds(sbase, SPT)],
                        out_hbm.at[pl.ds((c * 3 + q) * NSEG + sbase, SPT)])


_segment_sums = pl.kernel(
    _sc_body,
    out_type=jax.ShapeDtypeStruct((NC * 3 * NSEG, L), jnp.float32),
    mesh=plsc.VectorSubcoreMesh(core_axis_name="c", subcore_axis_name="s"),
    compiler_params=pltpu.CompilerParams(use_tc_tiling_on_sc=False,
                                         needs_layout_passes=False),
    scratch_types=[
        pltpu.VMEM((PIECE,), jnp.float32),
        pltpu.VMEM((PIECE,), jnp.float32),
        pltpu.VMEM((PIECE,), jnp.float32),
        pltpu.VMEM((PIECE,), jnp.float32),
        pltpu.VMEM((PIECE,), jnp.int32),
        pltpu.VMEM((PIECE,), jnp.int32),
        pltpu.VMEM((NSEG, L), jnp.float32),
        pltpu.VMEM((NSEG, L), jnp.float32),
        pltpu.VMEM((NSEG, L), jnp.float32),
        pltpu.VMEM((NSEG,), jnp.int32),
        pltpu.VMEM_SHARED((NSEG, L), jnp.float32),
        pltpu.VMEM_SHARED((NSEG, L), jnp.float32),
        pltpu.VMEM_SHARED((NSEG, L), jnp.float32),
        pltpu.SemaphoreType.DMA,
        pltpu.SemaphoreType.DMA,
    ],
)


def kernel(preds, target, batch_map):
    parts = _segment_sums(preds, target, batch_map.astype(jnp.int32))
    parts = parts.reshape(NC, 3, NSEG, L).sum(axis=(0, 3))
    pp, tt, pt = parts[0], parts[1], parts[2]
    eps = 1e-12
    cosine = pt / jnp.maximum(jnp.sqrt(pp) * jnp.sqrt(tt), eps)
    return jnp.mean(1.0 - cosine)
